# Initial kernel scaffold; baseline (speedup 1.0000x reference)
import jax
import jax.numpy as jnp
from jax.experimental import pallas as pl


def kernel(feat, edge, degree, W):
    raise NotImplementedError("write your pallas kernel here")


# speedup vs baseline: 4.6032x; 4.6032x over previous
import functools

import jax
import jax.numpy as jnp
from jax import lax
from jax.experimental import pallas as pl
from jax.experimental.pallas import tpu as pltpu
from jax.experimental.pallas import tpu_sc as plsc

NC = 2
NS = 16
NW = NC * NS
K = 128


def _sc_aggregate(feat, src, dst, zeros, *, ch, npad):
    n, d = feat.shape
    stripe = npad // NS

    def body(feat_hbm, src_hbm, dst_hbm, zeros_hbm, out_hbm,
             idx_s, idx_d, rows, agg_sh, gsem):
        c = lax.axis_index("c")
        s = lax.axis_index("s")
        wid = s * NC + c

        pltpu.sync_copy(zeros_hbm.at[pl.ds(s * stripe, stripe)],
                        agg_sh.at[pl.ds(s * stripe, stripe)])

        pltpu.sync_copy(src_hbm.at[wid], idx_s)
        pltpu.sync_copy(dst_hbm.at[wid], idx_d)
        plsc.subcore_barrier()

        def chunk(i, carry):
            pltpu.async_copy(feat_hbm.at[idx_s.at[i]], rows, gsem).wait()
            pltpu.sync_copy(rows, agg_sh.at[idx_d.at[i]], add=True)
            return carry

        lax.fori_loop(0, ch, chunk, 0)
        plsc.subcore_barrier()

        pltpu.sync_copy(agg_sh.at[pl.ds(s * stripe, stripe)],
                        out_hbm.at[c, pl.ds(s * stripe, stripe)])

    run = pl.kernel(
        body,
        out_type=jax.ShapeDtypeStruct((NC, npad, d), jnp.float32),
        mesh=plsc.VectorSubcoreMesh(core_axis_name="c", subcore_axis_name="s"),
        scratch_types=[
            pltpu.VMEM((ch, K), jnp.int32),
            pltpu.VMEM((ch, K), jnp.int32),
            pltpu.VMEM((K, d), jnp.float32),
            pltpu.VMEM_SHARED((npad, d), jnp.float32),
            pltpu.SemaphoreType.DMA,
        ],
    )
    return run(feat, src, dst, zeros)


def _tc_finish(partials, feat, degree2d, W, *, bn):
    n, d = feat.shape

    def body(p_ref, feat_ref, deg_ref, w_ref, o_ref):
        agg = p_ref[0] + p_ref[1]
        deg = deg_ref[...]
        inv = jnp.where(deg == 0.0, 1.0, 1.0 / deg)
        agg = agg * inv
        h = (jnp.dot(agg, w_ref[:d, :], preferred_element_type=jnp.float32)
             + jnp.dot(feat_ref[...], w_ref[d:, :],
                       preferred_element_type=jnp.float32))
        h = jnp.maximum(h, 0.0)
        denom = jnp.maximum(
            jnp.sqrt(jnp.sum(h * h, axis=1, keepdims=True)), 1e-12)
        o_ref[...] = h / denom

    return pl.pallas_call(
        body,
        grid=(n // bn,),
        in_specs=[
            pl.BlockSpec((NC, bn, d), lambda i: (0, i, 0)),
            pl.BlockSpec((bn, d), lambda i: (i, 0)),
            pl.BlockSpec((bn, 1), lambda i: (i, 0)),
            pl.BlockSpec((2 * d, d), lambda i: (0, 0)),
        ],
        out_specs=pl.BlockSpec((bn, d), lambda i: (i, 0)),
        out_shape=jax.ShapeDtypeStruct((n, d), jnp.float32),
    )(partials, feat, degree2d, W)


def kernel(feat, edge, degree, W):
    n, d = feat.shape
    e = edge.shape[0]

    ch = -(-e // (NW * K))
    epad = NW * K * ch
    npad = -(-(n + 1) // (NS * 8)) * (NS * 8)

    src = edge[:, 1].astype(jnp.int32)
    dst = edge[:, 0].astype(jnp.int32)
    pad = epad - e
    if pad:
        src = jnp.concatenate([src, jnp.zeros((pad,), jnp.int32)])
        dst = jnp.concatenate([dst, jnp.full((pad,), n, jnp.int32)])
    src = src.reshape(NW, ch, K)
    dst = dst.reshape(NW, ch, K)
    zeros = jnp.zeros((npad, d), jnp.float32)

    partials = _sc_aggregate(feat, src, dst, zeros, ch=ch, npad=npad)

    bn = 400 if n % 400 == 0 else 8
    return _tc_finish(partials, feat, degree.reshape(n, 1), W, bn=bn)
